# baseline (speedup 1.0000x reference)
import functools

import jax
import jax.numpy as jnp
from jax import lax
from jax.experimental import pallas as pl
from jax.experimental.pallas import tpu as pltpu
from jax.experimental.pallas import tpu_sc as plsc

NC = 2
NS = 16
NW = NC * NS
CHUNK = 128
SR = 20


def _ru(x, m):
    return (x + m - 1) // m * m


def _make_sc_aggregate(N, C, NSEG, RPT):
    NPAD = RPT * NS
    mesh = plsc.VectorSubcoreMesh(core_axis_name="c", subcore_axis_name="s")
    vt_last = N - (NS - 1) * RPT

    @functools.partial(
        pl.kernel,
        out_type=jax.ShapeDtypeStruct((NC, N, C), jnp.float32),
        mesh=mesh,
        scratch_types=[
            pltpu.VMEM((2, SR, CHUNK), jnp.int32),
            pltpu.VMEM((2, SR, CHUNK), jnp.int32),
            pltpu.VMEM((2, CHUNK, C), jnp.float32),
            pltpu.VMEM_SHARED((NPAD, C), jnp.float32),
            pltpu.SemaphoreType.DMA,
            pltpu.SemaphoreType.DMA,
            pltpu.SemaphoreType.DMA,
        ],
    )
    def sc_agg(m_hbm, sidx_hbm, ridx_hbm, zeros_hbm, out_hbm,
               sidx_v, ridx_v, rows_v, agg_sh, sem0, sem1, isem):
        cid = lax.axis_index("c")
        sid = lax.axis_index("s")
        wid = cid * NS + sid

        def iload(seg, b):
            pltpu.async_copy(sidx_hbm.at[wid, seg], sidx_v.at[b], isem)
            pltpu.async_copy(ridx_hbm.at[wid, seg], ridx_v.at[b], isem)

        def iwait(seg, b):
            pltpu.make_async_copy(sidx_hbm.at[wid, seg], sidx_v.at[b],
                                  isem).wait()
            pltpu.make_async_copy(ridx_hbm.at[wid, seg], ridx_v.at[b],
                                  isem).wait()

        iload(0, 0)
        pltpu.sync_copy(zeros_hbm, agg_sh.at[pl.ds(sid * RPT, RPT)])
        iwait(0, 0)
        plsc.subcore_barrier()

        def gstart(b, j, buf, sem):
            pltpu.async_copy(m_hbm.at[sidx_v.at[b].at[j]], rows_v.at[buf],
                             sem)

        def gwait(b, j, buf, sem):
            pltpu.make_async_copy(m_hbm.at[sidx_v.at[b].at[j]],
                                  rows_v.at[buf], sem).wait()

        def scat(b, j, buf):
            pltpu.sync_copy(rows_v.at[buf], agg_sh.at[ridx_v.at[b].at[j]],
                            add=True)

        for seg in range(NSEG):
            b = seg % 2
            has_next = seg + 1 < NSEG
            if has_next:
                iload(seg + 1, 1 - b)
            if seg == 0:
                gstart(0, 0, 0, sem0)

            def body(jj, carry, b=b):
                j0 = 2 * jj
                j1 = j0 + 1
                gwait(b, j0, 0, sem0)
                gstart(b, j1, 1, sem1)
                scat(b, j0, 0)
                gwait(b, j1, 1, sem1)
                gstart(b, j1 + 1, 0, sem0)
                scat(b, j1, 1)
                return carry

            lax.fori_loop(0, SR // 2 - 1, body, 0, unroll=False)

            j0, j1 = SR - 2, SR - 1
            gwait(b, j0, 0, sem0)
            gstart(b, j1, 1, sem1)
            scat(b, j0, 0)
            if has_next:
                iwait(seg + 1, 1 - b)
                gstart(1 - b, 0, 0, sem0)
            gwait(b, j1, 1, sem1)
            scat(b, j1, 1)
        plsc.subcore_barrier()

        @pl.when(sid < NS - 1)
        def _():
            pltpu.sync_copy(agg_sh.at[pl.ds(sid * RPT, RPT)],
                            out_hbm.at[cid].at[pl.ds(sid * RPT, RPT)])

        @pl.when(sid == NS - 1)
        def _():
            pltpu.sync_copy(agg_sh.at[pl.ds((NS - 1) * RPT, vt_last)],
                            out_hbm.at[cid].at[pl.ds((NS - 1) * RPT, vt_last)])

    return sc_agg


def _mm_body(x_ref, w_ref, o_ref):
    o_ref[...] = jnp.dot(x_ref[...], w_ref[...],
                         preferred_element_type=jnp.float32)


def _matmul(x, w, bn):
    n, c = x.shape
    k = w.shape[1]
    return pl.pallas_call(
        _mm_body,
        grid=(n // bn,),
        in_specs=[pl.BlockSpec((bn, c), lambda i: (i, 0)),
                  pl.BlockSpec((c, k), lambda i: (0, 0))],
        out_specs=pl.BlockSpec((bn, k), lambda i: (i, 0)),
        out_shape=jax.ShapeDtypeStruct((n, k), jnp.float32),
    )(x, w)


def _make_gru(N, C, BN, with_next):
    def body(x_ref, agg_ref, wi_ref, wh_ref, b_ref, *rest):
        if with_next:
            wn_ref, xo_ref, mo_ref = rest
        else:
            (xo_ref,) = rest
        x = x_ref[...]
        agg = agg_ref[0] + agg_ref[1]
        b = b_ref[0:1, :]
        gx = jnp.dot(agg, wi_ref[...], preferred_element_type=jnp.float32)
        zr_h = jnp.dot(x, wh_ref[:, :2 * C],
                       preferred_element_type=jnp.float32)
        zr = jax.nn.sigmoid(gx[:, :2 * C] + zr_h + b[:, :2 * C])
        z = zr[:, :C]
        r = zr[:, C:]
        a_h = jnp.dot(r * x, wh_ref[:, 2 * C:],
                      preferred_element_type=jnp.float32)
        a = jnp.tanh(gx[:, 2 * C:] + a_h + b[:, 2 * C:])
        xn = (1.0 - z) * x + z * a
        xo_ref[...] = xn
        if with_next:
            mo_ref[...] = jnp.dot(xn, wn_ref[...],
                                  preferred_element_type=jnp.float32)

    in_specs = [
        pl.BlockSpec((BN, C), lambda i: (i, 0)),
        pl.BlockSpec((NC, BN, C), lambda i: (0, i, 0)),
        pl.BlockSpec((C, 3 * C), lambda i: (0, 0)),
        pl.BlockSpec((C, 3 * C), lambda i: (0, 0)),
        pl.BlockSpec((8, 3 * C), lambda i: (0, 0)),
    ]
    out_specs = [pl.BlockSpec((BN, C), lambda i: (i, 0))]
    out_shape = [jax.ShapeDtypeStruct((N, C), jnp.float32)]
    if with_next:
        in_specs.append(pl.BlockSpec((C, C), lambda i: (0, 0)))
        out_specs.append(pl.BlockSpec((BN, C), lambda i: (i, 0)))
        out_shape.append(jax.ShapeDtypeStruct((N, C), jnp.float32))

    return pl.pallas_call(
        body,
        grid=(N // BN,),
        in_specs=in_specs,
        out_specs=out_specs,
        out_shape=out_shape,
    )


def kernel(nodes, senders, receivers, weights, w_i, w_h, b):
    N, C = nodes.shape
    L = weights.shape[0]
    E = senders.shape[0]

    NSEG = _ru(E, CHUNK * NW * SR) // (CHUNK * NW * SR)
    CH = NSEG * SR
    Epad = CH * CHUNK * NW
    RPT = _ru((N + 1 + NS - 1) // NS, 8)
    NPAD = RPT * NS
    if Epad > E:
        pad = Epad - E
        rr = jnp.arange(pad, dtype=jnp.int32)
        s_p = jnp.concatenate([senders, rr % N])
        r_p = jnp.concatenate([receivers, N + rr % (NPAD - N)])
    else:
        s_p, r_p = senders, receivers
    sidx = s_p.reshape(NW, NSEG, SR, CHUNK)
    ridx = r_p.reshape(NW, NSEG, SR, CHUNK)
    zeros_blk = jnp.zeros((RPT, C), jnp.float32)
    b2 = jnp.broadcast_to(b.reshape(1, 3 * C), (8, 3 * C))

    sc_agg = _make_sc_aggregate(N, C, NSEG, RPT)
    BN = 10000
    gru_mid = _make_gru(N, C, BN, with_next=True)
    gru_last = _make_gru(N, C, BN, with_next=False)

    x = nodes
    m = _matmul(x, weights[0], BN)
    for i in range(L):
        aggs = sc_agg(m, sidx, ridx, zeros_blk)
        if i + 1 < L:
            x, m = gru_mid(x, aggs, w_i, w_h, b2, weights[i + 1])
        else:
            (x,) = gru_last(x, aggs, w_i, w_h, b2)
    return x


# speedup vs baseline: 1.0138x; 1.0138x over previous
import functools

import jax
import jax.numpy as jnp
from jax import lax
from jax.experimental import pallas as pl
from jax.experimental.pallas import tpu as pltpu
from jax.experimental.pallas import tpu_sc as plsc

NC = 2
NS = 16
NW = NC * NS
CHUNK = 128
SR = 20


def _ru(x, m):
    return (x + m - 1) // m * m


def _make_sc_aggregate(N, C, NSEG, RPT):
    NPAD = RPT * NS
    mesh = plsc.VectorSubcoreMesh(core_axis_name="c", subcore_axis_name="s")
    vt_last = N - (NS - 1) * RPT

    @functools.partial(
        pl.kernel,
        out_type=jax.ShapeDtypeStruct((NC, N, C), jnp.float32),
        mesh=mesh,
        scratch_types=[
            pltpu.VMEM((2, SR, CHUNK), jnp.int32),
            pltpu.VMEM((2, SR, CHUNK), jnp.int32),
            pltpu.VMEM((2, CHUNK, C), jnp.float32),
            pltpu.VMEM_SHARED((NPAD, C), jnp.float32),
            pltpu.SemaphoreType.DMA,
            pltpu.SemaphoreType.DMA,
            pltpu.SemaphoreType.DMA,
        ],
    )
    def sc_agg(m_hbm, sidx_hbm, ridx_hbm, zeros_hbm, out_hbm,
               sidx_v, ridx_v, rows_v, agg_sh, sem0, sem1, isem):
        cid = lax.axis_index("c")
        sid = lax.axis_index("s")
        wid = cid * NS + sid

        def iload(seg, b):
            pltpu.async_copy(sidx_hbm.at[wid, seg], sidx_v.at[b], isem)
            pltpu.async_copy(ridx_hbm.at[wid, seg], ridx_v.at[b], isem)

        def iwait(seg, b):
            pltpu.make_async_copy(sidx_hbm.at[wid, seg], sidx_v.at[b],
                                  isem).wait()
            pltpu.make_async_copy(ridx_hbm.at[wid, seg], ridx_v.at[b],
                                  isem).wait()

        iload(0, 0)
        pltpu.sync_copy(zeros_hbm, agg_sh.at[pl.ds(sid * RPT, RPT)])
        iwait(0, 0)
        plsc.subcore_barrier()

        def gstart(b, j, buf, sem):
            pltpu.async_copy(m_hbm.at[sidx_v.at[b].at[j]], rows_v.at[buf],
                             sem)

        def gwait(b, j, buf, sem):
            pltpu.make_async_copy(m_hbm.at[sidx_v.at[b].at[j]],
                                  rows_v.at[buf], sem).wait()

        def scat(b, j, buf):
            pltpu.sync_copy(rows_v.at[buf], agg_sh.at[ridx_v.at[b].at[j]],
                            add=True)

        for seg in range(NSEG):
            b = seg % 2
            has_next = seg + 1 < NSEG
            if has_next:
                iload(seg + 1, 1 - b)
            if seg == 0:
                gstart(0, 0, 0, sem0)

            def body(jj, carry, b=b):
                j0 = 2 * jj
                j1 = j0 + 1
                gwait(b, j0, 0, sem0)
                gstart(b, j1, 1, sem1)
                scat(b, j0, 0)
                gwait(b, j1, 1, sem1)
                gstart(b, j1 + 1, 0, sem0)
                scat(b, j1, 1)
                return carry

            lax.fori_loop(0, SR // 2 - 1, body, 0, unroll=False)

            j0, j1 = SR - 2, SR - 1
            gwait(b, j0, 0, sem0)
            gstart(b, j1, 1, sem1)
            scat(b, j0, 0)
            if has_next:
                iwait(seg + 1, 1 - b)
                gstart(1 - b, 0, 0, sem0)
            gwait(b, j1, 1, sem1)
            scat(b, j1, 1)
        plsc.subcore_barrier()

        @pl.when(sid < NS - 1)
        def _():
            pltpu.sync_copy(agg_sh.at[pl.ds(sid * RPT, RPT)],
                            out_hbm.at[cid].at[pl.ds(sid * RPT, RPT)])

        @pl.when(sid == NS - 1)
        def _():
            pltpu.sync_copy(agg_sh.at[pl.ds((NS - 1) * RPT, vt_last)],
                            out_hbm.at[cid].at[pl.ds((NS - 1) * RPT, vt_last)])

    return sc_agg


def _mm_body(x_ref, w_ref, o_ref):
    o_ref[...] = jnp.dot(x_ref[...], w_ref[...],
                         preferred_element_type=jnp.float32)


def _matmul(x, w, bn):
    n, c = x.shape
    k = w.shape[1]
    return pl.pallas_call(
        _mm_body,
        grid=(n // bn,),
        in_specs=[pl.BlockSpec((bn, c), lambda i: (i, 0)),
                  pl.BlockSpec((c, k), lambda i: (0, 0))],
        out_specs=pl.BlockSpec((bn, k), lambda i: (i, 0)),
        out_shape=jax.ShapeDtypeStruct((n, k), jnp.float32),
    )(x, w)


def _make_gru(N, C, BN, with_next):
    def body(x_ref, agg_ref, wi_ref, wh_ref, b_ref, *rest):
        if with_next:
            wn_ref, xo_ref, mo_ref = rest
        else:
            (xo_ref,) = rest
        x = x_ref[...]
        agg = agg_ref[0] + agg_ref[1]
        b = b_ref[0:1, :]
        gx = jnp.dot(agg, wi_ref[...], preferred_element_type=jnp.float32)
        zr_h = jnp.dot(x, wh_ref[:, :2 * C],
                       preferred_element_type=jnp.float32)
        zr = jax.nn.sigmoid(gx[:, :2 * C] + zr_h + b[:, :2 * C])
        z = zr[:, :C]
        r = zr[:, C:]
        a_h = jnp.dot(r * x, wh_ref[:, 2 * C:],
                      preferred_element_type=jnp.float32)
        a = jnp.tanh(gx[:, 2 * C:] + a_h + b[:, 2 * C:])
        xn = (1.0 - z) * x + z * a
        xo_ref[...] = xn
        if with_next:
            mo_ref[...] = jnp.dot(xn, wn_ref[...],
                                  preferred_element_type=jnp.float32)

    in_specs = [
        pl.BlockSpec((BN, C), lambda i: (i, 0)),
        pl.BlockSpec((NC, BN, C), lambda i: (0, i, 0)),
        pl.BlockSpec((C, 3 * C), lambda i: (0, 0)),
        pl.BlockSpec((C, 3 * C), lambda i: (0, 0)),
        pl.BlockSpec((8, 3 * C), lambda i: (0, 0)),
    ]
    out_specs = [pl.BlockSpec((BN, C), lambda i: (i, 0))]
    out_shape = [jax.ShapeDtypeStruct((N, C), jnp.float32)]
    if with_next:
        in_specs.append(pl.BlockSpec((C, C), lambda i: (0, 0)))
        out_specs.append(pl.BlockSpec((BN, C), lambda i: (i, 0)))
        out_shape.append(jax.ShapeDtypeStruct((N, C), jnp.float32))

    return pl.pallas_call(
        body,
        grid=(N // BN,),
        in_specs=in_specs,
        out_specs=out_specs,
        out_shape=out_shape,
    )


def kernel(nodes, senders, receivers, weights, w_i, w_h, b):
    N, C = nodes.shape
    L = weights.shape[0]
    E = senders.shape[0]

    NSEG = _ru(E, CHUNK * NW * SR) // (CHUNK * NW * SR)
    CH = NSEG * SR
    Epad = CH * CHUNK * NW
    RPT = _ru((N + 1 + NS - 1) // NS, 8)
    NPAD = RPT * NS
    if Epad > E:
        pad = Epad - E
        rr = jnp.arange(pad, dtype=jnp.int32)
        s_p = jnp.concatenate([senders, rr % N])
        r_p = jnp.concatenate([receivers, N + rr % (NPAD - N)])
    else:
        s_p, r_p = senders, receivers
    sidx = s_p.reshape(NW, NSEG, SR, CHUNK)
    ridx = r_p.reshape(NW, NSEG, SR, CHUNK)
    zeros_blk = jnp.zeros((RPT, C), jnp.float32)
    b2 = jnp.broadcast_to(b.reshape(1, 3 * C), (8, 3 * C))

    sc_agg = _make_sc_aggregate(N, C, NSEG, RPT)
    BN = 5000
    gru_mid = _make_gru(N, C, BN, with_next=True)
    gru_last = _make_gru(N, C, BN, with_next=False)

    x = nodes
    m = _matmul(x, weights[0], BN)
    for i in range(L):
        aggs = sc_agg(m, sidx, ridx, zeros_blk)
        if i + 1 < L:
            x, m = gru_mid(x, aggs, w_i, w_h, b2, weights[i + 1])
        else:
            (x,) = gru_last(x, aggs, w_i, w_h, b2)
    return x
